# Initial kernel scaffold; baseline (speedup 1.0000x reference)
import jax
import jax.numpy as jnp
from jax.experimental import pallas as pl


def kernel(cat_idx, table):
    raise NotImplementedError("write your pallas kernel here")


# speedup vs baseline: 1.4363x; 1.4363x over previous
import functools

import jax
import jax.numpy as jnp
from jax import lax
from jax.experimental import pallas as pl
from jax.experimental.pallas import tpu as pltpu
from jax.experimental.pallas import tpu_sc as plsc

_NC, _NS = 2, 16
_NW = _NC * _NS
_IW = 128


@functools.partial(jax.jit, static_argnums=(2, 3))
def _sc_gather(idx2d, table, n_rows, d):
    rpw = n_rows // _NW

    mesh = plsc.VectorSubcoreMesh(
        core_axis_name="c", subcore_axis_name="s",
        num_cores=_NC, num_subcores=_NS)

    @functools.partial(
        pl.kernel,
        out_type=jax.ShapeDtypeStruct((n_rows, _IW, d), jnp.float32),
        mesh=mesh,
        scratch_types=[
            pltpu.VMEM((rpw, _IW), jnp.int32),
            pltpu.VMEM((_IW, d), jnp.float32),
            pltpu.SemaphoreType.DMA,
        ],
        compiler_params=pltpu.CompilerParams(use_tc_tiling_on_sc=False),
    )
    def k(idx_hbm, table_hbm, out_hbm, idx_v, rows_v, gsem):
        wid = lax.axis_index("s") * _NC + lax.axis_index("c")
        r0 = wid * rpw
        pltpu.sync_copy(idx_hbm.at[pl.ds(r0, rpw)], idx_v)

        def body(c, carry):
            pltpu.async_copy(
                table_hbm.at[idx_v.at[c]], rows_v, gsem
            ).wait()
            pltpu.sync_copy(rows_v, out_hbm.at[r0 + c])
            return carry

        lax.fori_loop(0, rpw, body, 0)

    return k(idx2d, table)


def kernel(cat_idx, table):
    s0, s1 = cat_idx.shape
    d = table.shape[1]
    n = s0 * s1
    n_rows = n // _IW
    idx2d = cat_idx.reshape(n_rows, _IW).astype(jnp.int32)
    out = _sc_gather(idx2d, table, n_rows, d)
    return out.reshape(s0, s1, d)


# speedup vs baseline: 1.5656x; 1.0900x over previous
import functools

import jax
import jax.numpy as jnp
from jax import lax
from jax.experimental import pallas as pl
from jax.experimental.pallas import tpu as pltpu
from jax.experimental.pallas import tpu_sc as plsc

_NC, _NS = 2, 16
_NW = _NC * _NS
_IW = 128


@functools.partial(jax.jit, static_argnums=(2, 3))
def _sc_gather(idx2d, table, n_rows, d):
    rpw = n_rows // _NW
    nbuf = 4
    nouter = rpw // nbuf

    mesh = plsc.VectorSubcoreMesh(
        core_axis_name="c", subcore_axis_name="s",
        num_cores=_NC, num_subcores=_NS)

    @functools.partial(
        pl.kernel,
        out_type=jax.ShapeDtypeStruct((n_rows, _IW, d), jnp.float32),
        mesh=mesh,
        scratch_types=[
            pltpu.VMEM((rpw, _IW), jnp.int32),
            pltpu.VMEM((nbuf, _IW, d), jnp.float32),
            [pltpu.SemaphoreType.DMA] * nbuf,
            [pltpu.SemaphoreType.DMA] * nbuf,
        ],
        compiler_params=pltpu.CompilerParams(use_tc_tiling_on_sc=False),
    )
    def k(idx_hbm, table_hbm, out_hbm, idx_v, rows_v, gsem, wsem):
        wid = lax.axis_index("s") * _NC + lax.axis_index("c")
        r0 = wid * rpw
        pltpu.sync_copy(idx_hbm.at[pl.ds(r0, rpw)], idx_v)

        for c in range(2):
            pltpu.async_copy(table_hbm.at[idx_v.at[c]], rows_v.at[c], gsem[c])

        def body(i, carry):
            for b in range(nbuf):
                c = i * nbuf + b
                bg = (b + 2) % nbuf

                @pl.when(c >= 2)
                def _():
                    pltpu.make_async_copy(
                        rows_v.at[bg], out_hbm.at[r0 + c - 2], wsem[bg]
                    ).wait()

                @pl.when(c + 2 < rpw)
                def _():
                    pltpu.async_copy(
                        table_hbm.at[idx_v.at[c + 2]], rows_v.at[bg], gsem[bg])

                pltpu.make_async_copy(
                    table_hbm.at[idx_v.at[c]], rows_v.at[b], gsem[b]
                ).wait()
                pltpu.async_copy(rows_v.at[b], out_hbm.at[r0 + c], wsem[b])
            return carry

        lax.fori_loop(0, nouter, body, 0)

        for b in (2, 3):
            c = (nouter - 1) * nbuf + b
            pltpu.make_async_copy(
                rows_v.at[b], out_hbm.at[r0 + c], wsem[b]
            ).wait()

    return k(idx2d, table)


def kernel(cat_idx, table):
    s0, s1 = cat_idx.shape
    d = table.shape[1]
    n = s0 * s1
    n_rows = n // _IW
    idx2d = cat_idx.reshape(n_rows, _IW).astype(jnp.int32)
    out = _sc_gather(idx2d, table, n_rows, d)
    return out.reshape(s0, s1, d)
